# baseline (speedup 1.0000x reference)
import functools

import jax
import jax.numpy as jnp
from jax import lax
from jax.experimental import pallas as pl
from jax.experimental.pallas import tpu as pltpu
from jax.experimental.pallas import tpu_sc as plsc

_LANES = 16


def _sc_grid(total: int):
    info = plsc.get_sparse_core_info()
    nc, ns = info.num_cores, info.num_subcores
    return nc, ns, nc * ns


@functools.lru_cache(maxsize=None)
def _build_packed_gather(total: int, split: int, factor: int, dim: int):
    nc, ns, nw = _sc_grid(total)
    assert total % nw == 0
    bpw = total // nw
    csz = 208
    while bpw % csz:
        csz //= 2
    nchunk = bpw // csz
    ngrp = csz // _LANES
    assert csz % _LANES == 0 and csz % 8 == 0
    pdim = factor * dim

    mesh = plsc.VectorSubcoreMesh(core_axis_name="core", subcore_axis_name="subcore")

    @functools.partial(
        pl.kernel,
        mesh=mesh,
        out_type=jax.ShapeDtypeStruct((total, dim), jnp.float32),
        compiler_params=pltpu.CompilerParams(
            use_tc_tiling_on_sc=False, needs_layout_passes=False
        ),
        scratch_types=[
            pltpu.VMEM((nchunk, csz), jnp.int32),
            pltpu.VMEM((csz, pdim), jnp.float32),
            pltpu.VMEM((csz, pdim), jnp.float32),
            pltpu.VMEM((csz, dim), jnp.float32),
            pltpu.VMEM((csz, dim), jnp.float32),
            pltpu.VMEM((csz,), jnp.int32),
            pltpu.VMEM((csz,), jnp.int32),
            pltpu.VMEM((csz,), jnp.int32),
            pltpu.VMEM((csz,), jnp.int32),
            pltpu.SemaphoreType.DMA,
            pltpu.SemaphoreType.DMA,
            pltpu.SemaphoreType.DMA,
            pltpu.SemaphoreType.DMA,
        ],
    )
    def gather(
        table_hbm, idx_hbm, out_hbm, idx_v,
        rows0, rows1, st0, st1, ir0, ir1, ov0, ov1, gs0, gs1, ws0, ws1,
    ):
        rows = (rows0, rows1)
        st = (st0, st1)
        ir = (ir0, ir1)
        ov = (ov0, ov1)
        gs = (gs0, gs1)
        ws = (ws0, ws1)
        wid = lax.axis_index("subcore") * nc + lax.axis_index("core")
        base = wid * bpw
        pltpu.sync_copy(idx_hbm.at[wid], idx_v)

        def prep(j, b):
            @pl.loop(0, ngrp)
            def _(g):
                sl = pl.ds(g * _LANES, _LANES)
                iv = idx_v[j, sl]
                f = (iv >= split).astype(jnp.int32)
                for k in range(2, factor):
                    f = f + (iv >= k * split).astype(jnp.int32)
                ir[b][sl] = iv - f * split
                ov[b][sl] = f * dim

        def extract(b):
            for g in range(ngrp):
                row16 = g * _LANES + lax.iota(jnp.int32, _LANES)
                off16 = ov[b][pl.ds(g * _LANES, _LANES)]
                for d in range(dim):
                    vals = plsc.load_gather(rows[b], [row16, off16 + d])
                    plsc.store_scatter(
                        st[b], [row16, jnp.full((_LANES,), d, jnp.int32)], vals
                    )

        @pl.loop(0, nchunk)
        def _(j):
            @pl.loop(0, ngrp)
            def _(g):
                sl = pl.ds(g * _LANES, _LANES)
                iv = idx_v[j, sl]
                one = jnp.full((_LANES,), 1, jnp.int32)
                zero = jnp.full((_LANES,), 0, jnp.int32)
                f = jnp.where(iv >= split, one, zero)
                for k in range(2, factor):
                    f = f + jnp.where(iv >= k * split, one, zero)
                ir[0][sl] = iv - f * split
                ov[0][sl] = f * dim

            pltpu.async_copy(table_hbm.at[ir[0]], rows[0], gs[0]).wait()
            extract(0)
            pltpu.sync_copy(st[0], out_hbm.at[pl.ds(base + j * csz, csz)])

    return gather


@functools.lru_cache(maxsize=None)
def _build_flat_gather(total: int, dim: int):
    nc, ns, nw = _sc_grid(total)
    assert total % nw == 0
    bpw = total // nw
    nchunk = 8
    while bpw % nchunk:
        nchunk += 1
    csz = bpw // nchunk
    assert csz % 8 == 0
    mesh = plsc.VectorSubcoreMesh(core_axis_name="core", subcore_axis_name="subcore")

    @functools.partial(
        pl.kernel,
        mesh=mesh,
        out_type=jax.ShapeDtypeStruct((total, dim), jnp.float32),
        compiler_params=pltpu.CompilerParams(use_tc_tiling_on_sc=False),
        scratch_types=[
            pltpu.VMEM((nchunk, csz), jnp.int32),
            pltpu.VMEM((csz, dim), jnp.float32),
            pltpu.VMEM((csz, dim), jnp.float32),
            pltpu.SemaphoreType.DMA,
            pltpu.SemaphoreType.DMA,
            pltpu.SemaphoreType.DMA,
            pltpu.SemaphoreType.DMA,
        ],
    )
    def gather(table_hbm, idx_hbm, out_hbm, idx_v, rows0, rows1, gs0, gs1, ws0, ws1):
        bufs = (rows0, rows1)
        gsems = (gs0, gs1)
        wsems = (ws0, ws1)
        wid = lax.axis_index("subcore") * nc + lax.axis_index("core")
        base = wid * bpw
        pltpu.sync_copy(idx_hbm.at[wid], idx_v)

        def gstart(j):
            return pltpu.async_copy(
                table_hbm.at[idx_v.at[j]], bufs[j % 2], gsems[j % 2]
            )

        def wstart(j):
            return pltpu.async_copy(
                bufs[j % 2], out_hbm.at[pl.ds(base + j * csz, csz)], wsems[j % 2]
            )

        gh = [None] * nchunk
        wh = [None] * nchunk
        gh[0] = gstart(0)
        if nchunk > 1:
            gh[1] = gstart(1)
        for j in range(nchunk):
            gh[j].wait()
            wh[j] = wstart(j)
            if j + 2 < nchunk:
                wh[j].wait()
                gh[j + 2] = gstart(j + 2)
        for j in range(max(0, nchunk - 2), nchunk):
            wh[j].wait()

    return gather


def kernel(indices, tables):
    b, s = indices.shape
    factor, split, dim = tables.shape
    total = b * s
    _, _, nw = _sc_grid(total)
    idx = indices.reshape(-1).astype(jnp.int32)
    if factor * dim == 128 and total % nw == 0 and (total // nw) % 16 == 0:
        fn = _build_packed_gather(total, split, factor, dim)
        bpw = total // nw
        csz = 208
        while bpw % csz:
            csz //= 2
        nchunk = bpw // csz
        table_v = tables.transpose(0, 2, 1).reshape(factor * dim, split).T
        out = fn(table_v, idx.reshape(nw, nchunk, csz))
    else:
        fn = _build_flat_gather(total, dim)
        bpw = total // nw
        nchunk = 8
        while bpw % nchunk:
            nchunk += 1
        csz = bpw // nchunk
        out = fn(tables.reshape(factor * split, dim), idx.reshape(nw, nchunk, csz))
    return out.reshape(b, s, dim)


# speedup vs baseline: 2.5046x; 2.5046x over previous
import functools

import jax
import jax.numpy as jnp
from jax import lax
from jax.experimental import pallas as pl
from jax.experimental.pallas import tpu as pltpu
from jax.experimental.pallas import tpu_sc as plsc

_LANES = 16


def _sc_grid(total: int):
    info = plsc.get_sparse_core_info()
    nc, ns = info.num_cores, info.num_subcores
    return nc, ns, nc * ns


@functools.lru_cache(maxsize=None)
def _build_packed_gather(total: int, split: int, factor: int, dim: int):
    nc, ns, nw = _sc_grid(total)
    assert total % nw == 0
    bpw = total // nw
    nchunk = 16
    while bpw % nchunk:
        nchunk += 1
    csz = bpw // nchunk
    ngrp = csz // _LANES
    assert csz % _LANES == 0 and csz % 8 == 0

    mesh = plsc.VectorSubcoreMesh(core_axis_name="core", subcore_axis_name="subcore")

    @functools.partial(
        pl.kernel,
        mesh=mesh,
        out_type=jax.ShapeDtypeStruct((total, dim), jnp.float32),
        compiler_params=pltpu.CompilerParams(
            use_tc_tiling_on_sc=False, needs_layout_passes=False
        ),
        scratch_types=[
            pltpu.VMEM((nchunk, csz), jnp.int32),
            pltpu.VMEM((csz, dim), jnp.float32),
            pltpu.VMEM((csz, dim), jnp.float32),
            pltpu.VMEM((csz,), jnp.int32),
            pltpu.VMEM((csz,), jnp.int32),
            pltpu.SemaphoreType.DMA,
            pltpu.SemaphoreType.DMA,
            pltpu.SemaphoreType.DMA,
            pltpu.SemaphoreType.DMA,
        ],
    )
    def gather(
        table_hbm, idx_hbm, out_hbm, idx_v,
        rows0, rows1, qr0, qr1, gs0, gs1, ws0, ws1,
    ):
        rows = (rows0, rows1)
        qr = (qr0, qr1)
        gs = (gs0, gs1)
        ws = (ws0, ws1)
        wid = lax.axis_index("subcore") * nc + lax.axis_index("core")
        base = wid * bpw
        pltpu.sync_copy(idx_hbm.at[wid], idx_v)

        def prep(j, b):
            @pl.loop(0, ngrp)
            def _(g):
                sl = pl.ds(g * _LANES, _LANES)
                iv = idx_v[j, sl]
                one = jnp.full((_LANES,), 1, jnp.int32)
                zero = jnp.full((_LANES,), 0, jnp.int32)
                f = jnp.where(iv >= split, one, zero)
                for k in range(2, factor):
                    f = f + jnp.where(iv >= k * split, one, zero)
                qr[b][sl] = (iv - f * split) * factor + f

        def gstart(j):
            return pltpu.async_copy(
                table_hbm.at[qr[j % 2]], rows[j % 2], gs[j % 2]
            )

        def wstart(j):
            return pltpu.async_copy(
                rows[j % 2], out_hbm.at[pl.ds(base + j * csz, csz)], ws[j % 2]
            )

        gh = [None] * nchunk
        wh = [None] * nchunk
        prep(0, 0)
        gh[0] = gstart(0)
        if nchunk > 1:
            prep(1, 1)
            gh[1] = gstart(1)
        for j in range(nchunk):
            gh[j].wait()
            wh[j] = wstart(j)
            if j + 2 < nchunk:
                wh[j].wait()
                prep(j + 2, j % 2)
                gh[j + 2] = gstart(j + 2)
        for j in range(max(0, nchunk - 2), nchunk):
            wh[j].wait()

    return gather


@functools.lru_cache(maxsize=None)
def _build_flat_gather(total: int, dim: int):
    nc, ns, nw = _sc_grid(total)
    assert total % nw == 0
    bpw = total // nw
    nchunk = 8
    while bpw % nchunk:
        nchunk += 1
    csz = bpw // nchunk
    assert csz % 8 == 0
    mesh = plsc.VectorSubcoreMesh(core_axis_name="core", subcore_axis_name="subcore")

    @functools.partial(
        pl.kernel,
        mesh=mesh,
        out_type=jax.ShapeDtypeStruct((total, dim), jnp.float32),
        compiler_params=pltpu.CompilerParams(use_tc_tiling_on_sc=False),
        scratch_types=[
            pltpu.VMEM((nchunk, csz), jnp.int32),
            pltpu.VMEM((csz, dim), jnp.float32),
            pltpu.VMEM((csz, dim), jnp.float32),
            pltpu.SemaphoreType.DMA,
            pltpu.SemaphoreType.DMA,
            pltpu.SemaphoreType.DMA,
            pltpu.SemaphoreType.DMA,
        ],
    )
    def gather(table_hbm, idx_hbm, out_hbm, idx_v, rows0, rows1, gs0, gs1, ws0, ws1):
        bufs = (rows0, rows1)
        gsems = (gs0, gs1)
        wsems = (ws0, ws1)
        wid = lax.axis_index("subcore") * nc + lax.axis_index("core")
        base = wid * bpw
        pltpu.sync_copy(idx_hbm.at[wid], idx_v)

        def gstart(j):
            return pltpu.async_copy(
                table_hbm.at[idx_v.at[j]], bufs[j % 2], gsems[j % 2]
            )

        def wstart(j):
            return pltpu.async_copy(
                bufs[j % 2], out_hbm.at[pl.ds(base + j * csz, csz)], wsems[j % 2]
            )

        gh = [None] * nchunk
        wh = [None] * nchunk
        gh[0] = gstart(0)
        if nchunk > 1:
            gh[1] = gstart(1)
        for j in range(nchunk):
            gh[j].wait()
            wh[j] = wstart(j)
            if j + 2 < nchunk:
                wh[j].wait()
                gh[j + 2] = gstart(j + 2)
        for j in range(max(0, nchunk - 2), nchunk):
            wh[j].wait()

    return gather


def kernel(indices, tables):
    b, s = indices.shape
    factor, split, dim = tables.shape
    total = b * s
    _, _, nw = _sc_grid(total)
    idx = indices.reshape(-1).astype(jnp.int32)
    if factor * dim == 128 and total % nw == 0 and (total // nw) % 16 == 0:
        fn = _build_packed_gather(total, split, factor, dim)
        bpw = total // nw
        nchunk = 16
        while bpw % nchunk:
            nchunk += 1
        csz = bpw // nchunk
        table_v = tables.transpose(0, 2, 1).reshape(factor * dim, split).T
        table_q = jax.lax.optimization_barrier(table_v).reshape(
            split * factor, dim
        )
        out = fn(table_q, idx.reshape(nw, nchunk, csz))
    else:
        fn = _build_flat_gather(total, dim)
        bpw = total // nw
        nchunk = 8
        while bpw % nchunk:
            nchunk += 1
        csz = bpw // nchunk
        out = fn(tables.reshape(factor * split, dim), idx.reshape(nw, nchunk, csz))
    return out.reshape(b, s, dim)


# speedup vs baseline: 2.5089x; 1.0017x over previous
import functools

import jax
import jax.numpy as jnp
from jax import lax
from jax.experimental import pallas as pl
from jax.experimental.pallas import tpu as pltpu
from jax.experimental.pallas import tpu_sc as plsc

_LANES = 16


def _sc_grid(total: int):
    info = plsc.get_sparse_core_info()
    nc, ns = info.num_cores, info.num_subcores
    return nc, ns, nc * ns


@functools.lru_cache(maxsize=None)
def _build_packed_gather(total: int, split: int, factor: int, dim: int):
    nc, ns, nw = _sc_grid(total)
    assert total % nw == 0
    bpw = total // nw
    nchunk = 8
    while bpw % nchunk:
        nchunk += 1
    csz = bpw // nchunk
    ngrp = csz // _LANES
    assert csz % _LANES == 0 and csz % 8 == 0

    mesh = plsc.VectorSubcoreMesh(core_axis_name="core", subcore_axis_name="subcore")

    @functools.partial(
        pl.kernel,
        mesh=mesh,
        out_type=jax.ShapeDtypeStruct((total, dim), jnp.float32),
        compiler_params=pltpu.CompilerParams(
            use_tc_tiling_on_sc=False, needs_layout_passes=False
        ),
        scratch_types=[
            pltpu.VMEM((nchunk, csz), jnp.int32),
            pltpu.VMEM((csz, dim), jnp.float32),
            pltpu.VMEM((csz, dim), jnp.float32),
            pltpu.VMEM((csz,), jnp.int32),
            pltpu.VMEM((csz,), jnp.int32),
            pltpu.SemaphoreType.DMA,
            pltpu.SemaphoreType.DMA,
            pltpu.SemaphoreType.DMA,
            pltpu.SemaphoreType.DMA,
        ],
    )
    def gather(
        table_hbm, idx_hbm, out_hbm, idx_v,
        rows0, rows1, qr0, qr1, gs0, gs1, ws0, ws1,
    ):
        rows = (rows0, rows1)
        qr = (qr0, qr1)
        gs = (gs0, gs1)
        ws = (ws0, ws1)
        wid = lax.axis_index("subcore") * nc + lax.axis_index("core")
        base = wid * bpw
        pltpu.sync_copy(idx_hbm.at[wid], idx_v)

        def prep(j, b):
            @pl.loop(0, ngrp)
            def _(g):
                sl = pl.ds(g * _LANES, _LANES)
                iv = idx_v[j, sl]
                one = jnp.full((_LANES,), 1, jnp.int32)
                zero = jnp.full((_LANES,), 0, jnp.int32)
                f = jnp.where(iv >= split, one, zero)
                for k in range(2, factor):
                    f = f + jnp.where(iv >= k * split, one, zero)
                qr[b][sl] = (iv - f * split) * factor + f

        def gstart(j):
            return pltpu.async_copy(
                table_hbm.at[qr[j % 2]], rows[j % 2], gs[j % 2]
            )

        def wstart(j):
            return pltpu.async_copy(
                rows[j % 2], out_hbm.at[pl.ds(base + j * csz, csz)], ws[j % 2]
            )

        gh = [None] * nchunk
        wh = [None] * nchunk
        prep(0, 0)
        gh[0] = gstart(0)
        if nchunk > 1:
            prep(1, 1)
            gh[1] = gstart(1)
        for j in range(nchunk):
            gh[j].wait()
            wh[j] = wstart(j)
            if j + 2 < nchunk:
                wh[j].wait()
                prep(j + 2, j % 2)
                gh[j + 2] = gstart(j + 2)
        for j in range(max(0, nchunk - 2), nchunk):
            wh[j].wait()

    return gather


@functools.lru_cache(maxsize=None)
def _build_flat_gather(total: int, dim: int):
    nc, ns, nw = _sc_grid(total)
    assert total % nw == 0
    bpw = total // nw
    nchunk = 8
    while bpw % nchunk:
        nchunk += 1
    csz = bpw // nchunk
    assert csz % 8 == 0
    mesh = plsc.VectorSubcoreMesh(core_axis_name="core", subcore_axis_name="subcore")

    @functools.partial(
        pl.kernel,
        mesh=mesh,
        out_type=jax.ShapeDtypeStruct((total, dim), jnp.float32),
        compiler_params=pltpu.CompilerParams(use_tc_tiling_on_sc=False),
        scratch_types=[
            pltpu.VMEM((nchunk, csz), jnp.int32),
            pltpu.VMEM((csz, dim), jnp.float32),
            pltpu.VMEM((csz, dim), jnp.float32),
            pltpu.SemaphoreType.DMA,
            pltpu.SemaphoreType.DMA,
            pltpu.SemaphoreType.DMA,
            pltpu.SemaphoreType.DMA,
        ],
    )
    def gather(table_hbm, idx_hbm, out_hbm, idx_v, rows0, rows1, gs0, gs1, ws0, ws1):
        bufs = (rows0, rows1)
        gsems = (gs0, gs1)
        wsems = (ws0, ws1)
        wid = lax.axis_index("subcore") * nc + lax.axis_index("core")
        base = wid * bpw
        pltpu.sync_copy(idx_hbm.at[wid], idx_v)

        def gstart(j):
            return pltpu.async_copy(
                table_hbm.at[idx_v.at[j]], bufs[j % 2], gsems[j % 2]
            )

        def wstart(j):
            return pltpu.async_copy(
                bufs[j % 2], out_hbm.at[pl.ds(base + j * csz, csz)], wsems[j % 2]
            )

        gh = [None] * nchunk
        wh = [None] * nchunk
        gh[0] = gstart(0)
        if nchunk > 1:
            gh[1] = gstart(1)
        for j in range(nchunk):
            gh[j].wait()
            wh[j] = wstart(j)
            if j + 2 < nchunk:
                wh[j].wait()
                gh[j + 2] = gstart(j + 2)
        for j in range(max(0, nchunk - 2), nchunk):
            wh[j].wait()

    return gather


def kernel(indices, tables):
    b, s = indices.shape
    factor, split, dim = tables.shape
    total = b * s
    _, _, nw = _sc_grid(total)
    idx = indices.reshape(-1).astype(jnp.int32)
    if factor * dim == 128 and total % nw == 0 and (total // nw) % 16 == 0:
        fn = _build_packed_gather(total, split, factor, dim)
        bpw = total // nw
        nchunk = 8
        while bpw % nchunk:
            nchunk += 1
        csz = bpw // nchunk
        table_v = tables.transpose(0, 2, 1).reshape(factor * dim, split).T
        table_q = jax.lax.optimization_barrier(table_v).reshape(
            split * factor, dim
        )
        out = fn(table_q, idx.reshape(nw, nchunk, csz))
    else:
        fn = _build_flat_gather(total, dim)
        bpw = total // nw
        nchunk = 8
        while bpw % nchunk:
            nchunk += 1
        csz = bpw // nchunk
        out = fn(tables.reshape(factor * split, dim), idx.reshape(nw, nchunk, csz))
    return out.reshape(b, s, dim)
